# baseline (speedup 1.0000x reference)
import functools

import jax
import jax.numpy as jnp
from jax import lax
from jax.experimental import pallas as pl
from jax.experimental.pallas import tpu as pltpu
from jax.experimental.pallas import tpu_sc as plsc

NC = 2
NS = 16
NW = NC * NS
NPAD = 10240
SROWS = NPAD // NS

KC = 100
NCH = 100
NBUF1, NBUF2 = 2, 4


def _zero_accumulator(zb_v, acc_sh, sid, k, d):
    zvec = jnp.zeros((16,), jnp.float32)

    @pl.loop(0, k)
    def _(r):
        @pl.loop(0, d, step=16)
        def _(col):
            zb_v[r, pl.ds(col, 16)] = zvec

    full, rem = SROWS // k, SROWS % k

    @pl.loop(0, full * k, step=k)
    def _(r0):
        pltpu.sync_copy(zb_v, acc_sh.at[pl.ds(sid * SROWS + r0, k)])

    if rem:
        pltpu.sync_copy(
            zb_v.at[pl.ds(0, rem)],
            acc_sh.at[pl.ds(sid * SROWS + full * k, rem)])


def _agg_loop(table_hbm, src_v, dst_v, rows, gsems, ssems, acc_sh, n_chunks):
    nbuf = len(rows)
    for s in range(nbuf):
        pltpu.async_copy(table_hbm.at[src_v.at[s]], rows[s], gsems[s])

    @pl.loop(0, n_chunks, step=nbuf)
    def _(j):
        for s in range(nbuf):
            pltpu.make_async_copy(
                table_hbm.at[src_v.at[j + s]], rows[s], gsems[s]).wait()
            pltpu.async_copy(
                rows[s], acc_sh.at[dst_v.at[j + s]], ssems[s], add=True)
        for s in range(nbuf):
            @pl.when(j + nbuf + s < n_chunks)
            def _(s=s):
                pltpu.make_async_copy(
                    rows[s], acc_sh.at[dst_v.at[j + s]], ssems[s]).wait()
                pltpu.async_copy(
                    table_hbm.at[src_v.at[j + nbuf + s]], rows[s], gsems[s])

    for s in range(nbuf):
        pltpu.make_async_copy(
            rows[s], acc_sh.at[dst_v.at[0]], ssems[s]).wait()


def _sc_agg_edgesplit(table, ei4, d, nbuf):
    mesh = plsc.VectorSubcoreMesh(core_axis_name="c", subcore_axis_name="s")

    @functools.partial(
        pl.kernel,
        out_type=jax.ShapeDtypeStruct((NC, NPAD, d), jnp.float32),
        mesh=mesh,
        scratch_types=[
            pltpu.VMEM((NCH, KC), jnp.int32),
            pltpu.VMEM((NCH, KC), jnp.int32),
        ] + [pltpu.VMEM((KC, d), jnp.float32) for _ in range(nbuf)] + [
            pltpu.VMEM_SHARED((NPAD, d), jnp.float32),
        ] + [pltpu.SemaphoreType.DMA for _ in range(2 * nbuf)],
        compiler_params=pltpu.CompilerParams(use_tc_tiling_on_sc=False),
    )
    def agg_kernel(table_hbm, ei_hbm, out_hbm,
                   src_v, dst_v, *bufs_and_sems):
        rows = bufs_and_sems[:nbuf]
        acc_sh = bufs_and_sems[nbuf]
        gsems = bufs_and_sems[nbuf + 1:2 * nbuf + 1]
        ssems = bufs_and_sems[2 * nbuf + 1:]
        cid = lax.axis_index("c")
        sid = lax.axis_index("s")
        wid = sid * NC + cid

        _zero_accumulator(rows[0], acc_sh, sid, KC, d)
        plsc.subcore_barrier()

        pltpu.sync_copy(ei_hbm.at[0, wid], src_v)
        pltpu.sync_copy(ei_hbm.at[1, wid], dst_v)
        _agg_loop(table_hbm, src_v, dst_v, rows, gsems, ssems,
                  acc_sh, NCH)

        plsc.subcore_barrier()
        pltpu.sync_copy(acc_sh.at[pl.ds(sid * SROWS, SROWS)],
                        out_hbm.at[cid, pl.ds(sid * SROWS, SROWS)])

    return agg_kernel(table, ei4)


def _tc_mlp(agg, w1, b1_2d, w2p, bm=1024):
    d_in = agg.shape[2]
    d_hid = w1.shape[1]
    d_out = w2p.shape[1]

    def body(a0, a1, w1r, b1r, w2r, o):
        full = a0[0] + a1[0]
        h = jnp.dot(full, w1r[...], preferred_element_type=jnp.float32)
        h = jnp.maximum(h + b1r[...], 0.0)
        o[...] = jnp.dot(h, w2r[...], preferred_element_type=jnp.float32)

    return pl.pallas_call(
        body,
        grid=(NPAD // bm,),
        in_specs=[
            pl.BlockSpec((1, bm, d_in), lambda i: (0, i, 0)),
            pl.BlockSpec((1, bm, d_in), lambda i: (1, i, 0)),
            pl.BlockSpec((d_in, d_hid), lambda i: (0, 0)),
            pl.BlockSpec((1, d_hid), lambda i: (0, 0)),
            pl.BlockSpec((d_hid, d_out), lambda i: (0, 0)),
        ],
        out_specs=pl.BlockSpec((bm, d_out), lambda i: (i, 0)),
        out_shape=jax.ShapeDtypeStruct((NPAD, d_out), jnp.float32),
    )(agg, agg, w1, b1_2d, w2p)


def _tc_final(acc, b2_2d, bm=1024):
    d = acc.shape[2]

    def body(a0, a1, b2r, o):
        o[...] = a0[0] + a1[0] + b2r[...]

    return pl.pallas_call(
        body,
        grid=(NPAD // bm,),
        in_specs=[
            pl.BlockSpec((1, bm, d), lambda i: (0, i, 0)),
            pl.BlockSpec((1, bm, d), lambda i: (1, i, 0)),
            pl.BlockSpec((1, d), lambda i: (0, 0)),
        ],
        out_specs=pl.BlockSpec((bm, d), lambda i: (i, 0)),
        out_shape=jax.ShapeDtypeStruct((NPAD, d), jnp.float32),
    )(acc, acc, b2_2d)


def kernel(x, edge_index, W1, b1, W2, b2):
    n_nodes, d_in = x.shape
    d_hid = W1.shape[1]
    d_out = W2.shape[1]
    d_out_pad = 48

    ei4 = edge_index.astype(jnp.int32).reshape(2, NW, NCH, KC)

    acc1 = _sc_agg_edgesplit(x, ei4, d_in, NBUF1)

    w2p = jnp.pad(W2, ((0, 0), (0, d_out_pad - d_out)))
    g = _tc_mlp(acc1, W1, b1.reshape(1, d_hid), w2p)

    acc2 = _sc_agg_edgesplit(g, ei4, d_out_pad, NBUF2)

    b2p = jnp.pad(b2, (0, d_out_pad - d_out)).reshape(1, d_out_pad)
    out = _tc_final(acc2, b2p)
    return out[:n_nodes, :d_out]


# speedup vs baseline: 1.0550x; 1.0550x over previous
import functools

import jax
import jax.numpy as jnp
from jax import lax
from jax.experimental import pallas as pl
from jax.experimental.pallas import tpu as pltpu
from jax.experimental.pallas import tpu_sc as plsc

NC = 2
NS = 16
NW = NC * NS
NPAD = 10240
SROWS = NPAD // NS

KC = 100
NCH = 100
K1, NBUF1 = 64, 3
NBUF2 = 4


def _zero_accumulator(zb_v, acc_sh, sid, k, d):
    zvec = jnp.zeros((16,), jnp.float32)

    @pl.loop(0, k)
    def _(r):
        @pl.loop(0, d, step=16)
        def _(col):
            zb_v[r, pl.ds(col, 16)] = zvec

    full, rem = SROWS // k, SROWS % k

    @pl.loop(0, full * k, step=k)
    def _(r0):
        pltpu.sync_copy(zb_v, acc_sh.at[pl.ds(sid * SROWS + r0, k)])

    if rem:
        pltpu.sync_copy(
            zb_v.at[pl.ds(0, rem)],
            acc_sh.at[pl.ds(sid * SROWS + full * k, rem)])


def _agg_loop(table_hbm, src_v, dst_v, rows, gsems, ssems, acc_sh, n_chunks):
    nbuf = len(rows)
    for s in range(nbuf):
        pltpu.async_copy(table_hbm.at[src_v.at[s]], rows[s], gsems[s])

    @pl.loop(0, n_chunks, step=nbuf)
    def _(j):
        for s in range(nbuf):
            pltpu.make_async_copy(
                table_hbm.at[src_v.at[j + s]], rows[s], gsems[s]).wait()
            pltpu.async_copy(
                rows[s], acc_sh.at[dst_v.at[j + s]], ssems[s], add=True)
        for s in range(nbuf):
            @pl.when(j + nbuf + s < n_chunks)
            def _(s=s):
                pltpu.make_async_copy(
                    rows[s], acc_sh.at[dst_v.at[j + s]], ssems[s]).wait()
                pltpu.async_copy(
                    table_hbm.at[src_v.at[j + nbuf + s]], rows[s], gsems[s])

    for s in range(nbuf):
        pltpu.make_async_copy(
            rows[s], acc_sh.at[dst_v.at[0]], ssems[s]).wait()


def _sc_agg_padded(table, src3, dst3, n_chunks, d, k, nbuf):
    mesh = plsc.VectorSubcoreMesh(core_axis_name="c", subcore_axis_name="s")

    @functools.partial(
        pl.kernel,
        out_type=jax.ShapeDtypeStruct((NC, NPAD, d), jnp.float32),
        mesh=mesh,
        scratch_types=[
            pltpu.VMEM((n_chunks, k), jnp.int32),
            pltpu.VMEM((n_chunks, k), jnp.int32),
        ] + [pltpu.VMEM((k, d), jnp.float32) for _ in range(nbuf)] + [
            pltpu.VMEM_SHARED((NPAD, d), jnp.float32),
        ] + [pltpu.SemaphoreType.DMA for _ in range(2 * nbuf)],
        compiler_params=pltpu.CompilerParams(use_tc_tiling_on_sc=False),
    )
    def agg_kernel(table_hbm, src_hbm, dst_hbm, out_hbm,
                   src_v, dst_v, *bufs_and_sems):
        rows = bufs_and_sems[:nbuf]
        acc_sh = bufs_and_sems[nbuf]
        gsems = bufs_and_sems[nbuf + 1:2 * nbuf + 1]
        ssems = bufs_and_sems[2 * nbuf + 1:]
        cid = lax.axis_index("c")
        sid = lax.axis_index("s")
        wid = sid * NC + cid

        _zero_accumulator(rows[0], acc_sh, sid, k, d)
        plsc.subcore_barrier()

        pltpu.sync_copy(src_hbm.at[wid], src_v)
        pltpu.sync_copy(dst_hbm.at[wid], dst_v)
        _agg_loop(table_hbm, src_v, dst_v, rows, gsems, ssems,
                  acc_sh, n_chunks)

        plsc.subcore_barrier()
        pltpu.sync_copy(acc_sh.at[pl.ds(sid * SROWS, SROWS)],
                        out_hbm.at[cid, pl.ds(sid * SROWS, SROWS)])

    return agg_kernel(table, src3, dst3)


def _pad_edges(src, dst, n_workers, n_nodes, k, nbuf):
    e_w = src.size // n_workers
    nch = -(-e_w // k)
    nch = -(-nch // nbuf) * nbuf
    pad = nch * k - e_w
    lane = jnp.arange(pad, dtype=jnp.int32)[None, :]
    wrk = jnp.arange(n_workers, dtype=jnp.int32)[:, None]
    fake_src = (lane * 131 + wrk * 613) % n_nodes
    fake_dst = n_nodes + (lane + wrk * 7) % (NPAD - n_nodes)
    src3 = jnp.concatenate([src.reshape(n_workers, e_w), fake_src], axis=1)
    dst3 = jnp.concatenate([dst.reshape(n_workers, e_w), fake_dst], axis=1)
    return (src3.reshape(n_workers, nch, k),
            dst3.reshape(n_workers, nch, k), nch)


def _sc_agg_edgesplit(table, ei4, d, nbuf):
    mesh = plsc.VectorSubcoreMesh(core_axis_name="c", subcore_axis_name="s")

    @functools.partial(
        pl.kernel,
        out_type=jax.ShapeDtypeStruct((NC, NPAD, d), jnp.float32),
        mesh=mesh,
        scratch_types=[
            pltpu.VMEM((NCH, KC), jnp.int32),
            pltpu.VMEM((NCH, KC), jnp.int32),
        ] + [pltpu.VMEM((KC, d), jnp.float32) for _ in range(nbuf)] + [
            pltpu.VMEM_SHARED((NPAD, d), jnp.float32),
        ] + [pltpu.SemaphoreType.DMA for _ in range(2 * nbuf)],
        compiler_params=pltpu.CompilerParams(use_tc_tiling_on_sc=False),
    )
    def agg_kernel(table_hbm, ei_hbm, out_hbm,
                   src_v, dst_v, *bufs_and_sems):
        rows = bufs_and_sems[:nbuf]
        acc_sh = bufs_and_sems[nbuf]
        gsems = bufs_and_sems[nbuf + 1:2 * nbuf + 1]
        ssems = bufs_and_sems[2 * nbuf + 1:]
        cid = lax.axis_index("c")
        sid = lax.axis_index("s")
        wid = sid * NC + cid

        _zero_accumulator(rows[0], acc_sh, sid, KC, d)
        plsc.subcore_barrier()

        pltpu.sync_copy(ei_hbm.at[0, wid], src_v)
        pltpu.sync_copy(ei_hbm.at[1, wid], dst_v)
        _agg_loop(table_hbm, src_v, dst_v, rows, gsems, ssems,
                  acc_sh, NCH)

        plsc.subcore_barrier()
        pltpu.sync_copy(acc_sh.at[pl.ds(sid * SROWS, SROWS)],
                        out_hbm.at[cid, pl.ds(sid * SROWS, SROWS)])

    return agg_kernel(table, ei4)


def _tc_mlp(agg, w1, b1_2d, w2p, bm=1024):
    d_in = agg.shape[2]
    d_hid = w1.shape[1]
    d_out = w2p.shape[1]

    def body(a0, a1, w1r, b1r, w2r, o):
        full = a0[0] + a1[0]
        h = jnp.dot(full, w1r[...], preferred_element_type=jnp.float32)
        h = jnp.maximum(h + b1r[...], 0.0)
        o[...] = jnp.dot(h, w2r[...], preferred_element_type=jnp.float32)

    return pl.pallas_call(
        body,
        grid=(NPAD // bm,),
        in_specs=[
            pl.BlockSpec((1, bm, d_in), lambda i: (0, i, 0)),
            pl.BlockSpec((1, bm, d_in), lambda i: (1, i, 0)),
            pl.BlockSpec((d_in, d_hid), lambda i: (0, 0)),
            pl.BlockSpec((1, d_hid), lambda i: (0, 0)),
            pl.BlockSpec((d_hid, d_out), lambda i: (0, 0)),
        ],
        out_specs=pl.BlockSpec((bm, d_out), lambda i: (i, 0)),
        out_shape=jax.ShapeDtypeStruct((NPAD, d_out), jnp.float32),
    )(agg, agg, w1, b1_2d, w2p)


def _tc_final(acc, b2_2d, bm=1024):
    d = acc.shape[2]

    def body(a0, a1, b2r, o):
        o[...] = a0[0] + a1[0] + b2r[...]

    return pl.pallas_call(
        body,
        grid=(NPAD // bm,),
        in_specs=[
            pl.BlockSpec((1, bm, d), lambda i: (0, i, 0)),
            pl.BlockSpec((1, bm, d), lambda i: (1, i, 0)),
            pl.BlockSpec((1, d), lambda i: (0, 0)),
        ],
        out_specs=pl.BlockSpec((bm, d), lambda i: (i, 0)),
        out_shape=jax.ShapeDtypeStruct((NPAD, d), jnp.float32),
    )(acc, acc, b2_2d)


def kernel(x, edge_index, W1, b1, W2, b2):
    n_nodes, d_in = x.shape
    d_hid = W1.shape[1]
    d_out = W2.shape[1]
    d_out_pad = 48

    ei = edge_index.astype(jnp.int32)
    ei4 = ei.reshape(2, NW, NCH, KC)

    src1, dst1, nch1 = _pad_edges(ei[0], ei[1], NW, n_nodes, K1, NBUF1)
    acc1 = _sc_agg_padded(x, src1, dst1, nch1, d_in, K1, NBUF1)

    w2p = jnp.pad(W2, ((0, 0), (0, d_out_pad - d_out)))
    g = _tc_mlp(acc1, W1, b1.reshape(1, d_hid), w2p)

    acc2 = _sc_agg_edgesplit(g, ei4, d_out_pad, NBUF2)

    b2p = jnp.pad(b2, (0, d_out_pad - d_out)).reshape(1, d_out_pad)
    out = _tc_final(acc2, b2p)
    return out[:n_nodes, :d_out]


# speedup vs baseline: 1.0899x; 1.0332x over previous
import functools

import jax
import jax.numpy as jnp
from jax import lax
from jax.experimental import pallas as pl
from jax.experimental.pallas import tpu as pltpu
from jax.experimental.pallas import tpu_sc as plsc

NC = 2
NS = 16
NW = NC * NS
NPAD = 10240
SROWS = NPAD // NS

K1, NBUF1 = 64, 3
K2, NBUF2 = 128, 4


def _zero_accumulator(zb_v, acc_sh, sid, k, d):
    zvec = jnp.zeros((16,), jnp.float32)

    @pl.loop(0, k)
    def _(r):
        @pl.loop(0, d, step=16)
        def _(col):
            zb_v[r, pl.ds(col, 16)] = zvec

    full, rem = SROWS // k, SROWS % k

    @pl.loop(0, full * k, step=k)
    def _(r0):
        pltpu.sync_copy(zb_v, acc_sh.at[pl.ds(sid * SROWS + r0, k)])

    if rem:
        pltpu.sync_copy(
            zb_v.at[pl.ds(0, rem)],
            acc_sh.at[pl.ds(sid * SROWS + full * k, rem)])


def _agg_loop(table_hbm, src_v, dst_v, rows, gsems, ssems, acc_sh, n_chunks):
    nbuf = len(rows)
    for s in range(nbuf):
        pltpu.async_copy(table_hbm.at[src_v.at[s]], rows[s], gsems[s])

    @pl.loop(0, n_chunks, step=nbuf)
    def _(j):
        for s in range(nbuf):
            pltpu.make_async_copy(
                table_hbm.at[src_v.at[j + s]], rows[s], gsems[s]).wait()
            pltpu.async_copy(
                rows[s], acc_sh.at[dst_v.at[j + s]], ssems[s], add=True)
        for s in range(nbuf):
            @pl.when(j + nbuf + s < n_chunks)
            def _(s=s):
                pltpu.make_async_copy(
                    rows[s], acc_sh.at[dst_v.at[j + s]], ssems[s]).wait()
                pltpu.async_copy(
                    table_hbm.at[src_v.at[j + nbuf + s]], rows[s], gsems[s])

    for s in range(nbuf):
        pltpu.make_async_copy(
            rows[s], acc_sh.at[dst_v.at[0]], ssems[s]).wait()


def _sc_agg_padded(table, src3, dst3, n_chunks, d, k, nbuf):
    mesh = plsc.VectorSubcoreMesh(core_axis_name="c", subcore_axis_name="s")

    @functools.partial(
        pl.kernel,
        out_type=jax.ShapeDtypeStruct((NC, NPAD, d), jnp.float32),
        mesh=mesh,
        scratch_types=[
            pltpu.VMEM((n_chunks, k), jnp.int32),
            pltpu.VMEM((n_chunks, k), jnp.int32),
        ] + [pltpu.VMEM((k, d), jnp.float32) for _ in range(nbuf)] + [
            pltpu.VMEM_SHARED((NPAD, d), jnp.float32),
        ] + [pltpu.SemaphoreType.DMA for _ in range(2 * nbuf + 2)],
        compiler_params=pltpu.CompilerParams(use_tc_tiling_on_sc=False),
    )
    def agg_kernel(table_hbm, src_hbm, dst_hbm, out_hbm,
                   src_v, dst_v, *bufs_and_sems):
        rows = bufs_and_sems[:nbuf]
        acc_sh = bufs_and_sems[nbuf]
        gsems = bufs_and_sems[nbuf + 1:2 * nbuf + 1]
        ssems = bufs_and_sems[2 * nbuf + 1:3 * nbuf + 1]
        isems = bufs_and_sems[3 * nbuf + 1:]
        cid = lax.axis_index("c")
        sid = lax.axis_index("s")
        wid = sid * NC + cid

        pltpu.async_copy(src_hbm.at[wid], src_v, isems[0])
        pltpu.async_copy(dst_hbm.at[wid], dst_v, isems[1])
        _zero_accumulator(rows[0], acc_sh, sid, k, d)
        plsc.subcore_barrier()

        pltpu.make_async_copy(src_hbm.at[wid], src_v, isems[0]).wait()
        pltpu.make_async_copy(dst_hbm.at[wid], dst_v, isems[1]).wait()
        _agg_loop(table_hbm, src_v, dst_v, rows, gsems, ssems,
                  acc_sh, n_chunks)

        plsc.subcore_barrier()
        pltpu.sync_copy(acc_sh.at[pl.ds(sid * SROWS, SROWS)],
                        out_hbm.at[cid, pl.ds(sid * SROWS, SROWS)])

    return agg_kernel(table, src3, dst3)


def _pad_edges(src, dst, n_workers, n_nodes, k, nbuf):
    e_w = src.size // n_workers
    nch = -(-e_w // k)
    nch = -(-nch // nbuf) * nbuf
    pad = nch * k - e_w
    lane = jnp.arange(pad, dtype=jnp.int32)[None, :]
    wrk = jnp.arange(n_workers, dtype=jnp.int32)[:, None]
    fake_src = (lane * 131 + wrk * 613) % n_nodes
    fake_dst = n_nodes + (lane + wrk * 7) % (NPAD - n_nodes)
    src3 = jnp.concatenate([src.reshape(n_workers, e_w), fake_src], axis=1)
    dst3 = jnp.concatenate([dst.reshape(n_workers, e_w), fake_dst], axis=1)
    return (src3.reshape(n_workers, nch, k),
            dst3.reshape(n_workers, nch, k), nch)


def _tc_mlp(agg, w1, b1_2d, w2p, bm=1024):
    d_in = agg.shape[2]
    d_hid = w1.shape[1]
    d_out = w2p.shape[1]

    def body(a0, a1, w1r, b1r, w2r, o):
        full = a0[0] + a1[0]
        h = jnp.dot(full, w1r[...], preferred_element_type=jnp.float32)
        h = jnp.maximum(h + b1r[...], 0.0)
        o[...] = jnp.dot(h, w2r[...], preferred_element_type=jnp.float32)

    return pl.pallas_call(
        body,
        grid=(NPAD // bm,),
        in_specs=[
            pl.BlockSpec((1, bm, d_in), lambda i: (0, i, 0)),
            pl.BlockSpec((1, bm, d_in), lambda i: (1, i, 0)),
            pl.BlockSpec((d_in, d_hid), lambda i: (0, 0)),
            pl.BlockSpec((1, d_hid), lambda i: (0, 0)),
            pl.BlockSpec((d_hid, d_out), lambda i: (0, 0)),
        ],
        out_specs=pl.BlockSpec((bm, d_out), lambda i: (i, 0)),
        out_shape=jax.ShapeDtypeStruct((NPAD, d_out), jnp.float32),
    )(agg, agg, w1, b1_2d, w2p)


def _tc_final(acc, b2_2d, bm=1024):
    d = acc.shape[2]

    def body(a0, a1, b2r, o):
        o[...] = a0[0] + a1[0] + b2r[...]

    return pl.pallas_call(
        body,
        grid=(NPAD // bm,),
        in_specs=[
            pl.BlockSpec((1, bm, d), lambda i: (0, i, 0)),
            pl.BlockSpec((1, bm, d), lambda i: (1, i, 0)),
            pl.BlockSpec((1, d), lambda i: (0, 0)),
        ],
        out_specs=pl.BlockSpec((bm, d), lambda i: (i, 0)),
        out_shape=jax.ShapeDtypeStruct((NPAD, d), jnp.float32),
    )(acc, acc, b2_2d)


def kernel(x, edge_index, W1, b1, W2, b2):
    n_nodes, d_in = x.shape
    d_hid = W1.shape[1]
    d_out = W2.shape[1]
    d_out_pad = 48

    ei = edge_index.astype(jnp.int32)

    src1, dst1, nch1 = _pad_edges(ei[0], ei[1], NW, n_nodes, K1, NBUF1)
    acc1 = _sc_agg_padded(x, src1, dst1, nch1, d_in, K1, NBUF1)

    w2p = jnp.pad(W2, ((0, 0), (0, d_out_pad - d_out)))
    g = _tc_mlp(acc1, W1, b1.reshape(1, d_hid), w2p)

    src2, dst2, nch2 = _pad_edges(ei[0], ei[1], NW, n_nodes, K2, NBUF2)
    acc2 = _sc_agg_padded(g, src2, dst2, nch2, d_out_pad, K2, NBUF2)

    b2p = jnp.pad(b2, (0, d_out_pad - d_out)).reshape(1, d_out_pad)
    out = _tc_final(acc2, b2p)
    return out[:n_nodes, :d_out]


# speedup vs baseline: 1.0904x; 1.0004x over previous
import functools

import jax
import jax.numpy as jnp
from jax import lax
from jax.experimental import pallas as pl
from jax.experimental.pallas import tpu as pltpu
from jax.experimental.pallas import tpu_sc as plsc

NC = 2
NS = 16
NW = NC * NS
NPAD = 10240
SROWS = NPAD // NS

K1, NBUF1 = 64, 3
K2, NBUF2 = 128, 6


def _zero_accumulator(zb_v, acc_sh, sid, k, d):
    zvec = jnp.zeros((16,), jnp.float32)

    @pl.loop(0, k)
    def _(r):
        @pl.loop(0, d, step=16)
        def _(col):
            zb_v[r, pl.ds(col, 16)] = zvec

    full, rem = SROWS // k, SROWS % k

    @pl.loop(0, full * k, step=k)
    def _(r0):
        pltpu.sync_copy(zb_v, acc_sh.at[pl.ds(sid * SROWS + r0, k)])

    if rem:
        pltpu.sync_copy(
            zb_v.at[pl.ds(0, rem)],
            acc_sh.at[pl.ds(sid * SROWS + full * k, rem)])


def _agg_loop(table_hbm, src_v, dst_v, rows, gsems, ssems, acc_sh, n_chunks):
    nbuf = len(rows)
    for s in range(nbuf):
        pltpu.async_copy(table_hbm.at[src_v.at[s]], rows[s], gsems[s])

    @pl.loop(0, n_chunks, step=nbuf)
    def _(j):
        for s in range(nbuf):
            pltpu.make_async_copy(
                table_hbm.at[src_v.at[j + s]], rows[s], gsems[s]).wait()
            pltpu.async_copy(
                rows[s], acc_sh.at[dst_v.at[j + s]], ssems[s], add=True)
        for s in range(nbuf):
            @pl.when(j + nbuf + s < n_chunks)
            def _(s=s):
                pltpu.make_async_copy(
                    rows[s], acc_sh.at[dst_v.at[j + s]], ssems[s]).wait()
                pltpu.async_copy(
                    table_hbm.at[src_v.at[j + nbuf + s]], rows[s], gsems[s])

    for s in range(nbuf):
        pltpu.make_async_copy(
            rows[s], acc_sh.at[dst_v.at[0]], ssems[s]).wait()


def _sc_agg_padded(table, src3, dst3, n_chunks, d, k, nbuf):
    mesh = plsc.VectorSubcoreMesh(core_axis_name="c", subcore_axis_name="s")

    @functools.partial(
        pl.kernel,
        out_type=jax.ShapeDtypeStruct((NC, NPAD, d), jnp.float32),
        mesh=mesh,
        scratch_types=[
            pltpu.VMEM((n_chunks, k), jnp.int32),
            pltpu.VMEM((n_chunks, k), jnp.int32),
        ] + [pltpu.VMEM((k, d), jnp.float32) for _ in range(nbuf)] + [
            pltpu.VMEM_SHARED((NPAD, d), jnp.float32),
        ] + [pltpu.SemaphoreType.DMA for _ in range(2 * nbuf + 2)],
        compiler_params=pltpu.CompilerParams(use_tc_tiling_on_sc=False),
    )
    def agg_kernel(table_hbm, src_hbm, dst_hbm, out_hbm,
                   src_v, dst_v, *bufs_and_sems):
        rows = bufs_and_sems[:nbuf]
        acc_sh = bufs_and_sems[nbuf]
        gsems = bufs_and_sems[nbuf + 1:2 * nbuf + 1]
        ssems = bufs_and_sems[2 * nbuf + 1:3 * nbuf + 1]
        isems = bufs_and_sems[3 * nbuf + 1:]
        cid = lax.axis_index("c")
        sid = lax.axis_index("s")
        wid = sid * NC + cid

        pltpu.async_copy(src_hbm.at[wid], src_v, isems[0])
        pltpu.async_copy(dst_hbm.at[wid], dst_v, isems[1])
        _zero_accumulator(rows[0], acc_sh, sid, k, d)
        plsc.subcore_barrier()

        pltpu.make_async_copy(src_hbm.at[wid], src_v, isems[0]).wait()
        pltpu.make_async_copy(dst_hbm.at[wid], dst_v, isems[1]).wait()
        _agg_loop(table_hbm, src_v, dst_v, rows, gsems, ssems,
                  acc_sh, n_chunks)

        plsc.subcore_barrier()
        pltpu.sync_copy(acc_sh.at[pl.ds(sid * SROWS, SROWS)],
                        out_hbm.at[cid, pl.ds(sid * SROWS, SROWS)])

    return agg_kernel(table, src3, dst3)


def _pad_edges(src, dst, n_workers, n_nodes, k, nbuf):
    e_w = src.size // n_workers
    nch = -(-e_w // k)
    nch = -(-nch // nbuf) * nbuf
    pad = nch * k - e_w
    lane = jnp.arange(pad, dtype=jnp.int32)[None, :]
    wrk = jnp.arange(n_workers, dtype=jnp.int32)[:, None]
    fake_src = (lane * 131 + wrk * 613) % n_nodes
    fake_dst = n_nodes + (lane + wrk * 7) % (NPAD - n_nodes)
    src3 = jnp.concatenate([src.reshape(n_workers, e_w), fake_src], axis=1)
    dst3 = jnp.concatenate([dst.reshape(n_workers, e_w), fake_dst], axis=1)
    return (src3.reshape(n_workers, nch, k),
            dst3.reshape(n_workers, nch, k), nch)


def _tc_mlp(agg, w1, b1_2d, w2p, bm=1024):
    d_in = agg.shape[2]
    d_hid = w1.shape[1]
    d_out = w2p.shape[1]

    def body(a0, a1, w1r, b1r, w2r, o):
        full = a0[0] + a1[0]
        h = jnp.dot(full, w1r[...], preferred_element_type=jnp.float32)
        h = jnp.maximum(h + b1r[...], 0.0)
        o[...] = jnp.dot(h, w2r[...], preferred_element_type=jnp.float32)

    return pl.pallas_call(
        body,
        grid=(NPAD // bm,),
        in_specs=[
            pl.BlockSpec((1, bm, d_in), lambda i: (0, i, 0)),
            pl.BlockSpec((1, bm, d_in), lambda i: (1, i, 0)),
            pl.BlockSpec((d_in, d_hid), lambda i: (0, 0)),
            pl.BlockSpec((1, d_hid), lambda i: (0, 0)),
            pl.BlockSpec((d_hid, d_out), lambda i: (0, 0)),
        ],
        out_specs=pl.BlockSpec((bm, d_out), lambda i: (i, 0)),
        out_shape=jax.ShapeDtypeStruct((NPAD, d_out), jnp.float32),
    )(agg, agg, w1, b1_2d, w2p)


def _tc_final(acc, b2_2d, bm=1024):
    d = acc.shape[2]

    def body(a0, a1, b2r, o):
        o[...] = a0[0] + a1[0] + b2r[...]

    return pl.pallas_call(
        body,
        grid=(NPAD // bm,),
        in_specs=[
            pl.BlockSpec((1, bm, d), lambda i: (0, i, 0)),
            pl.BlockSpec((1, bm, d), lambda i: (1, i, 0)),
            pl.BlockSpec((1, d), lambda i: (0, 0)),
        ],
        out_specs=pl.BlockSpec((bm, d), lambda i: (i, 0)),
        out_shape=jax.ShapeDtypeStruct((NPAD, d), jnp.float32),
    )(acc, acc, b2_2d)


def kernel(x, edge_index, W1, b1, W2, b2):
    n_nodes, d_in = x.shape
    d_hid = W1.shape[1]
    d_out = W2.shape[1]
    d_out_pad = 48

    ei = edge_index.astype(jnp.int32)

    src1, dst1, nch1 = _pad_edges(ei[0], ei[1], NW, n_nodes, K1, NBUF1)
    acc1 = _sc_agg_padded(x, src1, dst1, nch1, d_in, K1, NBUF1)

    w2p = jnp.pad(W2, ((0, 0), (0, d_out_pad - d_out)))
    g = _tc_mlp(acc1, W1, b1.reshape(1, d_hid), w2p)

    src2, dst2, nch2 = _pad_edges(ei[0], ei[1], NW, n_nodes, K2, NBUF2)
    acc2 = _sc_agg_padded(g, src2, dst2, nch2, d_out_pad, K2, NBUF2)

    b2p = jnp.pad(b2, (0, d_out_pad - d_out)).reshape(1, d_out_pad)
    out = _tc_final(acc2, b2p)
    return out[:n_nodes, :d_out]


# speedup vs baseline: 1.1030x; 1.0116x over previous
import functools

import jax
import jax.numpy as jnp
from jax import lax
from jax.experimental import pallas as pl
from jax.experimental.pallas import tpu as pltpu
from jax.experimental.pallas import tpu_sc as plsc

NC = 2
NS = 16
NW = NC * NS
NPAD = 10240
SROWS = NPAD // NS

K1, NBUF1 = 64, 3
K2, NBUF2 = 128, 6


def _zero_accumulator(zb_v, acc_sh, sid, k, d):
    zvec = jnp.zeros((16,), jnp.float32)

    @pl.loop(0, k)
    def _(r):
        @pl.loop(0, d, step=16)
        def _(col):
            zb_v[r, pl.ds(col, 16)] = zvec

    full, rem = SROWS // k, SROWS % k

    @pl.loop(0, full * k, step=k)
    def _(r0):
        pltpu.sync_copy(zb_v, acc_sh.at[pl.ds(sid * SROWS + r0, k)])

    if rem:
        pltpu.sync_copy(
            zb_v.at[pl.ds(0, rem)],
            acc_sh.at[pl.ds(sid * SROWS + full * k, rem)])


def _agg_loop(table_hbm, src_v, dst_v, rows, gsems, ssems, acc_sh, n_chunks):
    nbuf = len(rows)
    for s in range(nbuf):
        pltpu.async_copy(table_hbm.at[src_v.at[s]], rows[s], gsems[s])

    @pl.loop(0, n_chunks, step=nbuf)
    def _(j):
        for s in range(nbuf):
            pltpu.make_async_copy(
                table_hbm.at[src_v.at[j + s]], rows[s], gsems[s]).wait()
            pltpu.async_copy(
                rows[s], acc_sh.at[dst_v.at[j + s]], ssems[s], add=True)
        for s in range(nbuf):
            @pl.when(j + nbuf + s < n_chunks)
            def _(s=s):
                pltpu.make_async_copy(
                    rows[s], acc_sh.at[dst_v.at[j + s]], ssems[s]).wait()
                pltpu.async_copy(
                    table_hbm.at[src_v.at[j + nbuf + s]], rows[s], gsems[s])

    for s in range(nbuf):
        pltpu.make_async_copy(
            rows[s], acc_sh.at[dst_v.at[0]], ssems[s]).wait()


def _sc_agg_padded(table, src3, dst3, n_chunks, d, k, nbuf):
    mesh = plsc.VectorSubcoreMesh(core_axis_name="c", subcore_axis_name="s")

    @functools.partial(
        pl.kernel,
        out_type=jax.ShapeDtypeStruct((NC, NPAD, d), jnp.float32),
        mesh=mesh,
        scratch_types=[
            pltpu.VMEM((n_chunks, k), jnp.int32),
            pltpu.VMEM((n_chunks, k), jnp.int32),
        ] + [pltpu.VMEM((k, d), jnp.float32) for _ in range(nbuf)] + [
            pltpu.VMEM_SHARED((NPAD, d), jnp.float32),
        ] + [pltpu.SemaphoreType.DMA for _ in range(2 * nbuf + 2)],
        compiler_params=pltpu.CompilerParams(use_tc_tiling_on_sc=False),
    )
    def agg_kernel(table_hbm, src_hbm, dst_hbm, out_hbm,
                   src_v, dst_v, *bufs_and_sems):
        rows = bufs_and_sems[:nbuf]
        acc_sh = bufs_and_sems[nbuf]
        gsems = bufs_and_sems[nbuf + 1:2 * nbuf + 1]
        ssems = bufs_and_sems[2 * nbuf + 1:3 * nbuf + 1]
        isems = bufs_and_sems[3 * nbuf + 1:]
        cid = lax.axis_index("c")
        sid = lax.axis_index("s")
        wid = sid * NC + cid

        pltpu.async_copy(src_hbm.at[wid], src_v, isems[0])
        pltpu.async_copy(dst_hbm.at[wid], dst_v, isems[1])
        _zero_accumulator(rows[0], acc_sh, sid, k, d)
        plsc.subcore_barrier()

        pltpu.make_async_copy(src_hbm.at[wid], src_v, isems[0]).wait()
        pltpu.make_async_copy(dst_hbm.at[wid], dst_v, isems[1]).wait()
        _agg_loop(table_hbm, src_v, dst_v, rows, gsems, ssems,
                  acc_sh, n_chunks)

        plsc.subcore_barrier()
        pltpu.sync_copy(acc_sh.at[pl.ds(sid * SROWS, SROWS)],
                        out_hbm.at[cid, pl.ds(sid * SROWS, SROWS)])

    return agg_kernel(table, src3, dst3)


def _pad_edges(src, dst, n_workers, n_nodes, k, nbuf):
    e_w = src.size // n_workers
    nch = -(-e_w // k)
    nch = -(-nch // nbuf) * nbuf
    pad = nch * k - e_w
    lane = jnp.arange(pad, dtype=jnp.int32)[None, :]
    wrk = jnp.arange(n_workers, dtype=jnp.int32)[:, None]
    fake_src = (lane * 131 + wrk * 613) % n_nodes
    fake_dst = n_nodes + (lane + wrk * 7) % (NPAD - n_nodes)
    src3 = jnp.concatenate([src.reshape(n_workers, e_w), fake_src], axis=1)
    dst3 = jnp.concatenate([dst.reshape(n_workers, e_w), fake_dst], axis=1)
    return (src3.reshape(n_workers, nch, k),
            dst3.reshape(n_workers, nch, k), nch)


def _tc_mlp(agg, w1, b1_2d, w2p, bm=2048):
    d_in = agg.shape[2]
    d_hid = w1.shape[1]
    d_out = w2p.shape[1]

    def body(a0, a1, w1r, b1r, w2r, o):
        full = a0[0] + a1[0]
        h = jnp.dot(full, w1r[...], preferred_element_type=jnp.float32)
        h = jnp.maximum(h + b1r[...], 0.0)
        o[...] = jnp.dot(h, w2r[...], preferred_element_type=jnp.float32)

    return pl.pallas_call(
        body,
        grid=(NPAD // bm,),
        in_specs=[
            pl.BlockSpec((1, bm, d_in), lambda i: (0, i, 0)),
            pl.BlockSpec((1, bm, d_in), lambda i: (1, i, 0)),
            pl.BlockSpec((d_in, d_hid), lambda i: (0, 0)),
            pl.BlockSpec((1, d_hid), lambda i: (0, 0)),
            pl.BlockSpec((d_hid, d_out), lambda i: (0, 0)),
        ],
        out_specs=pl.BlockSpec((bm, d_out), lambda i: (i, 0)),
        out_shape=jax.ShapeDtypeStruct((NPAD, d_out), jnp.float32),
    )(agg, agg, w1, b1_2d, w2p)


def _tc_final(acc, b2_2d, bm=1024):
    d = acc.shape[2]

    def body(a0, a1, b2r, o):
        o[...] = a0[0] + a1[0] + b2r[...]

    return pl.pallas_call(
        body,
        grid=(NPAD // bm,),
        in_specs=[
            pl.BlockSpec((1, bm, d), lambda i: (0, i, 0)),
            pl.BlockSpec((1, bm, d), lambda i: (1, i, 0)),
            pl.BlockSpec((1, d), lambda i: (0, 0)),
        ],
        out_specs=pl.BlockSpec((bm, d), lambda i: (i, 0)),
        out_shape=jax.ShapeDtypeStruct((NPAD, d), jnp.float32),
    )(acc, acc, b2_2d)


def kernel(x, edge_index, W1, b1, W2, b2):
    n_nodes, d_in = x.shape
    d_hid = W1.shape[1]
    d_out = W2.shape[1]
    d_out_pad = 48

    ei = edge_index.astype(jnp.int32)

    src1, dst1, nch1 = _pad_edges(ei[0], ei[1], NW, n_nodes, K1, NBUF1)
    acc1 = _sc_agg_padded(x, src1, dst1, nch1, d_in, K1, NBUF1)

    w2p = jnp.pad(W2, ((0, 0), (0, d_out_pad - d_out)))
    g = _tc_mlp(acc1, W1, b1.reshape(1, d_hid), w2p)

    src2, dst2, nch2 = _pad_edges(ei[0], ei[1], NW, n_nodes, K2, NBUF2)
    acc2 = _sc_agg_padded(g, src2, dst2, nch2, d_out_pad, K2, NBUF2)

    b2p = jnp.pad(b2, (0, d_out_pad - d_out)).reshape(1, d_out_pad)
    out = _tc_final(acc2, b2p)
    return out[:n_nodes, :d_out]
